# baseline (speedup 1.0000x reference)
import functools

import jax
import jax.numpy as jnp
from jax import lax
from jax.experimental import pallas as pl
from jax.experimental.pallas import tpu as pltpu
from jax.experimental.pallas import tpu_sc as plsc

N_NODES = 10000
N_EDGES = 320000
D = 128
N_GRAPHS = 64

NC = 2
NS = 16
NW = NC * NS
CHUNK = 128
N_CHUNKS = N_EDGES // CHUNK
CH_PER_W = 80
PAD_CHUNKS = NW * CH_PER_W
ACC_ROWS = 10112
ROWS_PER_TILE = ACC_ROWS // NS


def _seg_sum_sc(h, src2, dst2):
    mesh = plsc.VectorSubcoreMesh(
        core_axis_name="c", subcore_axis_name="s", num_cores=NC, num_subcores=NS
    )

    @functools.partial(
        pl.kernel,
        out_type=jax.ShapeDtypeStruct((NC, ACC_ROWS, D), jnp.float32),
        mesh=mesh,
        scratch_types=[
            pltpu.VMEM((CH_PER_W // 2, CHUNK), jnp.int32),
            pltpu.VMEM((CH_PER_W // 2, CHUNK), jnp.int32),
            pltpu.VMEM((2, CHUNK, D), jnp.float32),
            pltpu.VMEM_SHARED((ACC_ROWS, D), jnp.float32),
            pltpu.SemaphoreType.DMA,
            pltpu.SemaphoreType.DMA,
        ],
    )
    def body(h_hbm, src_hbm, dst_hbm, out_hbm, src_v, dst_v, rows_v, acc,
             sem0, sem1):
        cid = lax.axis_index("c")
        sid = lax.axis_index("s")
        wid = sid * NC + cid
        sems = (sem0, sem1)
        n = CH_PER_W
        half = n // 2

        def load_idx(hi):
            base = wid * n + hi * half
            pltpu.sync_copy(src_hbm.at[pl.ds(base, half)], src_v)
            pltpu.sync_copy(dst_hbm.at[pl.ds(base, half)], dst_v)

        load_idx(0)

        def zrow(i, _):
            def zcol(k, _):
                rows_v[1, i, pl.ds(k * 16, 16)] = jnp.zeros((16,), jnp.float32)
                return 0

            lax.fori_loop(0, D // 16, zcol, 0)
            return 0

        lax.fori_loop(0, CHUNK, zrow, 0)

        zbase = sid * ROWS_PER_TILE
        n_full = ROWS_PER_TILE // CHUNK
        for t in range(n_full):
            pltpu.sync_copy(rows_v.at[1], acc.at[pl.ds(zbase + t * CHUNK, CHUNK)])
        rem = ROWS_PER_TILE - n_full * CHUNK
        if rem:
            pltpu.sync_copy(
                rows_v.at[1, pl.ds(0, rem)],
                acc.at[pl.ds(zbase + n_full * CHUNK, rem)],
            )

        plsc.subcore_barrier()

        def pipe(g, _):
            pltpu.async_copy(h_hbm.at[src_v.at[g]], rows_v.at[0], sem0).wait()
            pltpu.sync_copy(rows_v.at[0], acc.at[dst_v.at[g]], add=True)
            return 0

        lax.fori_loop(0, half, pipe, 0)

        load_idx(1)
        lax.fori_loop(0, half, pipe, 0)

        plsc.subcore_barrier()

        pltpu.sync_copy(
            acc.at[pl.ds(zbase, ROWS_PER_TILE)],
            out_hbm.at[cid, pl.ds(zbase, ROWS_PER_TILE)],
        )

    return body(h, src2, dst2)


_BLK = 1000
_GRID = N_NODES // _BLK


def _mlp_body(eps_s, h_ref, parts_ref, w1t, b1, w2t, b2, g, b, rm, rv, out_ref):
    t = (1.0 + eps_s[0, 0]) * h_ref[...] + parts_ref[0] + parts_ref[1]
    a = jnp.dot(t, w1t[...], preferred_element_type=jnp.float32) + b1[...]
    a = jnp.maximum(a, 0.0)
    a = jnp.dot(a, w2t[...], preferred_element_type=jnp.float32) + b2[...]
    a = jnp.maximum(a, 0.0)
    scale = g[...] * lax.rsqrt(rv[...] + 1e-5)
    out_ref[...] = a * scale + (b[...] - rm[...] * scale)


def _mlp_tc(h, parts, p):
    vec = lambda v: v.reshape(1, D)
    full = pl.BlockSpec((D, D), lambda i: (0, 0))
    vspec = pl.BlockSpec((1, D), lambda i: (0, 0))
    return pl.pallas_call(
        _mlp_body,
        grid=(_GRID,),
        in_specs=[
            pl.BlockSpec(memory_space=pltpu.SMEM),
            pl.BlockSpec((_BLK, D), lambda i: (i, 0)),
            pl.BlockSpec((NC, _BLK, D), lambda i: (0, i, 0)),
            full, vspec, full, vspec, vspec, vspec, vspec, vspec,
        ],
        out_specs=pl.BlockSpec((_BLK, D), lambda i: (i, 0)),
        out_shape=jax.ShapeDtypeStruct((N_NODES, D), jnp.float32),
    )(
        p["eps"].reshape(1, 1),
        h,
        parts,
        p["W1"].T,
        vec(p["b1"]),
        p["W2"].T,
        vec(p["b2"]),
        vec(p["bn_g"]),
        vec(p["bn_b"]),
        vec(p["bn_rm"]),
        vec(p["bn_rv"]),
    )


def _head_body(h_ref, batch_ref, w1t, b1, w2t, b2, out_ref):
    ids = lax.broadcasted_iota(jnp.int32, (N_NODES, N_GRAPHS), 1)
    m = (batch_ref[...] == ids).astype(jnp.float32)
    dn = (((0,), (0,)), ((), ()))
    sums = lax.dot_general(m, h_ref[...], dn, preferred_element_type=jnp.float32)
    counts = lax.dot_general(
        m, jnp.ones((N_NODES, 1), jnp.float32), dn,
        preferred_element_type=jnp.float32,
    )
    pooled = sums / jnp.maximum(counts, 1.0)
    a = jnp.dot(pooled, w1t[...], preferred_element_type=jnp.float32) + b1[...]
    a = jnp.maximum(a, 0.0)
    z = jnp.dot(a, w2t[...], preferred_element_type=jnp.float32) + b2[...]
    zmax = jnp.max(z, axis=1, keepdims=True)
    e = jnp.exp(z - zmax)
    lse = jnp.log(jnp.sum(e, axis=1, keepdims=True))
    out_ref[...] = z - zmax - lse


def _head_tc(h, batch, params):
    return pl.pallas_call(
        _head_body,
        out_shape=jax.ShapeDtypeStruct((N_GRAPHS, 10), jnp.float32),
    )(
        h,
        batch.reshape(N_NODES, 1),
        params["lin1_W"].T,
        params["lin1_b"].reshape(1, D),
        params["lin2_W"].T,
        params["lin2_b"].reshape(1, 10),
    )


def kernel(x, edge_index, batch, params):
    pad = (PAD_CHUNKS - N_CHUNKS) * CHUNK
    src = edge_index[0].astype(jnp.int32)
    dst = edge_index[1].astype(jnp.int32)
    src2 = jnp.concatenate([src, jnp.zeros((pad,), jnp.int32)])
    src2 = src2.reshape(PAD_CHUNKS, CHUNK)
    dst2 = jnp.concatenate([dst, jnp.full((pad,), N_NODES, jnp.int32)])
    dst2 = dst2.reshape(PAD_CHUNKS, CHUNK)
    h = x
    layer_params = [params["conv1"]] + list(params["convs"])
    for p in layer_params:
        parts = _seg_sum_sc(h, src2, dst2)
        h = _mlp_tc(h, parts, p)
    return _head_tc(h, batch.astype(jnp.int32), params)


# speedup vs baseline: 1.1266x; 1.1266x over previous
import functools

import jax
import jax.numpy as jnp
from jax import lax
from jax.experimental import pallas as pl
from jax.experimental.pallas import tpu as pltpu
from jax.experimental.pallas import tpu_sc as plsc

N_NODES = 10000
N_EDGES = 320000
D = 128
N_GRAPHS = 64

NC = 2
NS = 16
NW = NC * NS
CHUNK = 128
N_CHUNKS = N_EDGES // CHUNK
CH_PER_W = 80
PAD_CHUNKS = NW * CH_PER_W
ACC_ROWS = 10112
ROWS_PER_TILE = ACC_ROWS // NS


def _seg_sum_sc(h, idx2):
    mesh = plsc.VectorSubcoreMesh(
        core_axis_name="c", subcore_axis_name="s", num_cores=NC, num_subcores=NS
    )

    @functools.partial(
        pl.kernel,
        out_type=jax.ShapeDtypeStruct((NC, ACC_ROWS, D), jnp.float32),
        mesh=mesh,
        scratch_types=[
            pltpu.VMEM((2, CHUNK), jnp.int32),
            pltpu.VMEM((2, CHUNK), jnp.int32),
            pltpu.VMEM((2, CHUNK), jnp.int32),
            pltpu.VMEM((2, CHUNK), jnp.int32),
            pltpu.VMEM((CHUNK, D), jnp.float32),
            pltpu.VMEM((CHUNK, D), jnp.float32),
            pltpu.VMEM_SHARED((ACC_ROWS, D), jnp.float32),
            pltpu.SemaphoreType.DMA,
            pltpu.SemaphoreType.DMA,
            pltpu.SemaphoreType.DMA,
            pltpu.SemaphoreType.DMA,
        ],
    )
    def body(h_hbm, idx_hbm, out_hbm, i0, i1, i2, i3, r0, r1, acc,
             g0, g1, t0, t1):
        cid = lax.axis_index("c")
        sid = lax.axis_index("s")
        wid = sid * NC + cid
        ibufs = (i0, i1, i2, i3)
        rows, gsems, ssems = (r0, r1), (g0, g1), (t0, t1)
        n = CH_PER_W
        base = wid * n

        pltpu.sync_copy(idx_hbm.at[base], i0)
        pltpu.async_copy(h_hbm.at[i0.at[0]], r0, g0)

        def zrow(i, _):
            def zcol(k, _):
                r1[i, pl.ds(k * 16, 16)] = jnp.zeros((16,), jnp.float32)
                return 0

            lax.fori_loop(0, D // 16, zcol, 0)
            return 0

        lax.fori_loop(0, CHUNK, zrow, 0)

        zbase = sid * ROWS_PER_TILE
        n_full = ROWS_PER_TILE // CHUNK
        for t in range(n_full):
            pltpu.sync_copy(r1, acc.at[pl.ds(zbase + t * CHUNK, CHUNK)])
        rem = ROWS_PER_TILE - n_full * CHUNK
        if rem:
            pltpu.sync_copy(
                r1.at[pl.ds(0, rem)],
                acc.at[pl.ds(zbase + n_full * CHUNK, rem)],
            )

        plsc.subcore_barrier()

        def pipe(g, _):
            for b in range(4):
                j = 4 * g + b
                rb = b % 2
                nrb = 1 - rb
                nib = (b + 1) % 4
                pib = (b + 3) % 4

                @pl.when(j < n - 1)
                def _():
                    pltpu.sync_copy(idx_hbm.at[base + j + 1], ibufs[nib])

                    @pl.when(j >= 1)
                    def _():
                        pltpu.make_async_copy(
                            rows[nrb], acc.at[ibufs[pib].at[1]], ssems[nrb]
                        ).wait()

                    pltpu.async_copy(
                        h_hbm.at[ibufs[nib].at[0]], rows[nrb], gsems[nrb]
                    )

                pltpu.make_async_copy(
                    h_hbm.at[ibufs[b].at[0]], rows[rb], gsems[rb]
                ).wait()
                pltpu.async_copy(
                    rows[rb], acc.at[ibufs[b].at[1]], ssems[rb], add=True
                )
            return 0

        lax.fori_loop(0, n // 4, pipe, 0)

        pltpu.make_async_copy(r0, acc.at[i2.at[1]], t0).wait()
        pltpu.make_async_copy(r1, acc.at[i3.at[1]], t1).wait()

        plsc.subcore_barrier()

        pltpu.sync_copy(
            acc.at[pl.ds(zbase, ROWS_PER_TILE)],
            out_hbm.at[cid, pl.ds(zbase, ROWS_PER_TILE)],
        )

    return body(h, idx2)


_BLK = 1000
_GRID = N_NODES // _BLK


def _mlp_body(eps_s, h_ref, parts_ref, w1t, b1, w2t, b2, g, b, rm, rv, out_ref):
    t = (1.0 + eps_s[0, 0]) * h_ref[...] + parts_ref[0] + parts_ref[1]
    a = jnp.dot(t, w1t[...], preferred_element_type=jnp.float32) + b1[...]
    a = jnp.maximum(a, 0.0)
    a = jnp.dot(a, w2t[...], preferred_element_type=jnp.float32) + b2[...]
    a = jnp.maximum(a, 0.0)
    scale = g[...] * lax.rsqrt(rv[...] + 1e-5)
    out_ref[...] = a * scale + (b[...] - rm[...] * scale)


def _mlp_tc(h, parts, p):
    vec = lambda v: v.reshape(1, D)
    full = pl.BlockSpec((D, D), lambda i: (0, 0))
    vspec = pl.BlockSpec((1, D), lambda i: (0, 0))
    return pl.pallas_call(
        _mlp_body,
        grid=(_GRID,),
        in_specs=[
            pl.BlockSpec(memory_space=pltpu.SMEM),
            pl.BlockSpec((_BLK, D), lambda i: (i, 0)),
            pl.BlockSpec((NC, _BLK, D), lambda i: (0, i, 0)),
            full, vspec, full, vspec, vspec, vspec, vspec, vspec,
        ],
        out_specs=pl.BlockSpec((_BLK, D), lambda i: (i, 0)),
        out_shape=jax.ShapeDtypeStruct((N_NODES, D), jnp.float32),
    )(
        p["eps"].reshape(1, 1),
        h,
        parts,
        p["W1"].T,
        vec(p["b1"]),
        p["W2"].T,
        vec(p["b2"]),
        vec(p["bn_g"]),
        vec(p["bn_b"]),
        vec(p["bn_rm"]),
        vec(p["bn_rv"]),
    )


def _head_body(h_ref, batch_ref, w1t, b1, w2t, b2, out_ref):
    ids = lax.broadcasted_iota(jnp.int32, (N_NODES, N_GRAPHS), 1)
    m = (batch_ref[...] == ids).astype(jnp.float32)
    dn = (((0,), (0,)), ((), ()))
    sums = lax.dot_general(m, h_ref[...], dn, preferred_element_type=jnp.float32)
    counts = lax.dot_general(
        m, jnp.ones((N_NODES, 1), jnp.float32), dn,
        preferred_element_type=jnp.float32,
    )
    pooled = sums / jnp.maximum(counts, 1.0)
    a = jnp.dot(pooled, w1t[...], preferred_element_type=jnp.float32) + b1[...]
    a = jnp.maximum(a, 0.0)
    z = jnp.dot(a, w2t[...], preferred_element_type=jnp.float32) + b2[...]
    zmax = jnp.max(z, axis=1, keepdims=True)
    e = jnp.exp(z - zmax)
    lse = jnp.log(jnp.sum(e, axis=1, keepdims=True))
    out_ref[...] = z - zmax - lse


def _head_tc(h, batch, params):
    return pl.pallas_call(
        _head_body,
        out_shape=jax.ShapeDtypeStruct((N_GRAPHS, 10), jnp.float32),
    )(
        h,
        batch.reshape(N_NODES, 1),
        params["lin1_W"].T,
        params["lin1_b"].reshape(1, D),
        params["lin2_W"].T,
        params["lin2_b"].reshape(1, 10),
    )


def kernel(x, edge_index, batch, params):
    pad = (PAD_CHUNKS - N_CHUNKS) * CHUNK
    src = edge_index[0].astype(jnp.int32)
    dst = edge_index[1].astype(jnp.int32)
    src2 = jnp.concatenate([src, jnp.zeros((pad,), jnp.int32)])
    dst2 = jnp.concatenate([dst, jnp.full((pad,), N_NODES, jnp.int32)])
    idx2 = jnp.stack(
        [src2.reshape(PAD_CHUNKS, CHUNK), dst2.reshape(PAD_CHUNKS, CHUNK)],
        axis=1,
    )
    h = x
    layer_params = [params["conv1"]] + list(params["convs"])
    for p in layer_params:
        parts = _seg_sum_sc(h, idx2)
        h = _mlp_tc(h, parts, p)
    return _head_tc(h, batch.astype(jnp.int32), params)
